# baseline (speedup 1.0000x reference)
import jax
import jax.numpy as jnp
from jax.experimental import pallas as pl
from jax.experimental.pallas import tpu as pltpu

_CHUNK = 8


def _fused_kernel(x_ref, w0_ref, wr_ref, b_ref, wd_ref, bd_ref, out_ref,
                  g0_scr, hc_scr, h_scr, c_scr, acc_scr, whb_scr):
    c = pl.program_id(0)
    num_layers = b_ref.shape[0]
    hidden = b_ref.shape[2] // 4
    in_size = x_ref.shape[2]
    C = wd_ref.shape[0]

    @pl.when(c == 0)
    def _init():
        xs = x_ref[:, 0, :]
        g0_scr[...] = (jnp.dot(xs, w0_ref[0:in_size, :],
                               preferred_element_type=jnp.float32)
                       + b_ref[0])
        whb_scr[0] = w0_ref[in_size:in_size + hidden, :].astype(jnp.bfloat16)
        for l in range(1, num_layers):
            whb_scr[l] = wr_ref[l - 1, hidden:2 * hidden, :].astype(jnp.bfloat16)
        h_scr[...] = jnp.zeros_like(h_scr)
        c_scr[...] = jnp.zeros_like(c_scr)
        acc_scr[...] = jnp.zeros_like(acc_scr)

    def step_layer(gates, l):
        i_g = jax.nn.sigmoid(gates[:, 0 * hidden:1 * hidden])
        f_g = jax.nn.sigmoid(gates[:, 1 * hidden:2 * hidden])
        g_g = jnp.tanh(gates[:, 2 * hidden:3 * hidden])
        o_g = jax.nn.sigmoid(gates[:, 3 * hidden:4 * hidden])
        c_new = f_g * c_scr[l] + i_g * g_g
        h_new = o_g * jnp.tanh(c_new)
        c_scr[l] = c_new
        h_scr[l] = h_new
        return h_new

    base = c * C

    for j in range(C):
        gates = (g0_scr[pl.ds(base + j, 1), :]
                 + jnp.dot(h_scr[0].astype(jnp.bfloat16), whb_scr[0],
                           preferred_element_type=jnp.float32))
        hc_scr[j:j + 1, :] = step_layer(gates, 0)

    for l in range(1, num_layers):
        g_chunk = (jnp.dot(hc_scr[...], wr_ref[l - 1, 0:hidden, :],
                           preferred_element_type=jnp.float32)
                   + b_ref[l])
        for j in range(C):
            gates = (g_chunk[j:j + 1, :]
                     + jnp.dot(h_scr[l].astype(jnp.bfloat16), whb_scr[l],
                               preferred_element_type=jnp.float32))
            hc_scr[j:j + 1, :] = step_layer(gates, l)

    acc = acc_scr[...]
    for j in range(C):
        acc = acc + jnp.dot(hc_scr[j:j + 1, :], wd_ref[j],
                            preferred_element_type=jnp.float32)
    acc_scr[...] = acc

    @pl.when(c == pl.num_programs(0) - 1)
    def _finalize():
        out_ref[...] = jax.nn.sigmoid(acc_scr[...] + bd_ref[...]).astype(out_ref.dtype)


@jax.jit
def kernel(x, w0, wr, b_all, wd, bd):
    seq_len, _, in_size = x.shape
    num_layers = b_all.shape[0]
    hidden = b_all.shape[2] // 4
    out_size = wd.shape[2]
    lr = wr.shape[0]
    chunk = _CHUNK if seq_len % _CHUNK == 0 else 1

    return pl.pallas_call(
        _fused_kernel,
        out_shape=jax.ShapeDtypeStruct((1, out_size), jnp.float32),
        grid_spec=pltpu.PrefetchScalarGridSpec(
            num_scalar_prefetch=0,
            grid=(seq_len // chunk,),
            in_specs=[
                pl.BlockSpec((seq_len, 1, in_size), lambda c: (0, 0, 0)),
                pl.BlockSpec((in_size + hidden, 4 * hidden), lambda c: (0, 0)),
                pl.BlockSpec((lr, 2 * hidden, 4 * hidden), lambda c: (0, 0, 0)),
                pl.BlockSpec((num_layers, 1, 4 * hidden), lambda c: (0, 0, 0)),
                pl.BlockSpec((chunk, hidden, out_size), lambda c: (c, 0, 0)),
                pl.BlockSpec((1, out_size), lambda c: (0, 0)),
            ],
            out_specs=pl.BlockSpec((1, out_size), lambda c: (0, 0)),
            scratch_shapes=[
                pltpu.VMEM((seq_len, 4 * hidden), jnp.float32),
                pltpu.VMEM((chunk, hidden), jnp.float32),
                pltpu.VMEM((num_layers, 1, hidden), jnp.float32),
                pltpu.VMEM((num_layers, 1, hidden), jnp.float32),
                pltpu.VMEM((1, out_size), jnp.float32),
                pltpu.VMEM((num_layers, hidden, 4 * hidden), jnp.bfloat16),
            ],
        ),
        compiler_params=pltpu.CompilerParams(
            dimension_semantics=("arbitrary",)),
    )(x, w0, wr, b_all, wd, bd)


# speedup vs baseline: 1.1610x; 1.1610x over previous
import jax
import jax.numpy as jnp
from jax.experimental import pallas as pl
from jax.experimental.pallas import tpu as pltpu

_CHUNK = 8


def _fused_kernel(x_ref, w0_ref, wr_ref, b_ref, wd_ref, bd_ref, out_ref,
                  g0_scr, g1_scr, h0c_scr, hc_scr, h_scr, c_scr, acc_scr,
                  whb_scr):
    c = pl.program_id(0)
    nc = pl.num_programs(0) - 1
    num_layers = b_ref.shape[0]
    hidden = b_ref.shape[2] // 4
    in_size = x_ref.shape[2]
    C = wd_ref.shape[0]
    seq_len = x_ref.shape[0]

    @pl.when(c == 0)
    def _init():
        xs = x_ref[:, 0, :]
        g0_scr[0:seq_len, :] = (jnp.dot(xs, w0_ref[0:in_size, :],
                                        preferred_element_type=jnp.float32)
                                + b_ref[0])
        g0_scr[seq_len:seq_len + C, :] = jnp.zeros((C, 4 * hidden), jnp.float32)
        g1_scr[...] = jnp.zeros_like(g1_scr)
        whb_scr[0] = w0_ref[in_size:in_size + hidden, :].astype(jnp.bfloat16)
        for l in range(1, num_layers):
            whb_scr[l] = wr_ref[l - 1, hidden:2 * hidden, :].astype(jnp.bfloat16)
        h_scr[...] = jnp.zeros_like(h_scr)
        c_scr[...] = jnp.zeros_like(c_scr)
        acc_scr[...] = jnp.zeros_like(acc_scr)

    def step_layer(gates, l):
        i_g = jax.nn.sigmoid(gates[:, 0 * hidden:1 * hidden])
        f_g = jax.nn.sigmoid(gates[:, 1 * hidden:2 * hidden])
        g_g = jnp.tanh(gates[:, 2 * hidden:3 * hidden])
        o_g = jax.nn.sigmoid(gates[:, 3 * hidden:4 * hidden])
        c_new = f_g * c_scr[l] + i_g * g_g
        h_new = o_g * jnp.tanh(c_new)
        c_scr[l] = c_new
        h_scr[l] = h_new
        return h_new

    acc = acc_scr[...]
    for l in range(1, num_layers):
        if l == 1:
            g_chunk = g1_scr[...]
        else:
            g_chunk = (jnp.dot(hc_scr[...], wr_ref[l - 1, 0:hidden, :],
                               preferred_element_type=jnp.float32)
                       + b_ref[l])
        for j in range(C):
            gates = (g_chunk[j:j + 1, :]
                     + jnp.dot(h_scr[l].astype(jnp.bfloat16), whb_scr[l],
                               preferred_element_type=jnp.float32))
            h_new = step_layer(gates, l)
            hc_scr[j:j + 1, :] = h_new
            if l == num_layers - 1:
                acc = acc + jnp.dot(h_new, wd_ref[j],
                                    preferred_element_type=jnp.float32)
    acc_scr[...] = acc

    base = c * C
    for j in range(C):
        gates = (g0_scr[pl.ds(base + j, 1), :]
                 + jnp.dot(h_scr[0].astype(jnp.bfloat16), whb_scr[0],
                           preferred_element_type=jnp.float32))
        h0c_scr[j:j + 1, :] = step_layer(gates, 0)

    g1_scr[...] = (jnp.dot(h0c_scr[...], wr_ref[0, 0:hidden, :],
                           preferred_element_type=jnp.float32)
                   + b_ref[1])

    @pl.when(c == nc)
    def _finalize():
        out_ref[...] = jax.nn.sigmoid(acc_scr[...] + bd_ref[...]).astype(out_ref.dtype)


@jax.jit
def kernel(x, w0, wr, b_all, wd, bd):
    seq_len, _, in_size = x.shape
    num_layers = b_all.shape[0]
    hidden = b_all.shape[2] // 4
    out_size = wd.shape[2]
    lr = wr.shape[0]
    chunk = _CHUNK if seq_len % _CHUNK == 0 else 1
    nc = seq_len // chunk

    return pl.pallas_call(
        _fused_kernel,
        out_shape=jax.ShapeDtypeStruct((1, out_size), jnp.float32),
        grid_spec=pltpu.PrefetchScalarGridSpec(
            num_scalar_prefetch=0,
            grid=(nc + 1,),
            in_specs=[
                pl.BlockSpec((seq_len, 1, in_size), lambda c: (0, 0, 0)),
                pl.BlockSpec((in_size + hidden, 4 * hidden), lambda c: (0, 0)),
                pl.BlockSpec((lr, 2 * hidden, 4 * hidden), lambda c: (0, 0, 0)),
                pl.BlockSpec((num_layers, 1, 4 * hidden), lambda c: (0, 0, 0)),
                pl.BlockSpec((chunk, hidden, out_size),
                             lambda c: (jnp.maximum(c - 1, 0), 0, 0)),
                pl.BlockSpec((1, out_size), lambda c: (0, 0)),
            ],
            out_specs=pl.BlockSpec((1, out_size), lambda c: (0, 0)),
            scratch_shapes=[
                pltpu.VMEM((seq_len + chunk, 4 * hidden), jnp.float32),
                pltpu.VMEM((chunk, 4 * hidden), jnp.float32),
                pltpu.VMEM((chunk, hidden), jnp.float32),
                pltpu.VMEM((chunk, hidden), jnp.float32),
                pltpu.VMEM((num_layers, 1, hidden), jnp.float32),
                pltpu.VMEM((num_layers, 1, hidden), jnp.float32),
                pltpu.VMEM((1, out_size), jnp.float32),
                pltpu.VMEM((num_layers, hidden, 4 * hidden), jnp.bfloat16),
            ],
        ),
        compiler_params=pltpu.CompilerParams(
            dimension_semantics=("arbitrary",)),
    )(x, w0, wr, b_all, wd, bd)


# speedup vs baseline: 1.1679x; 1.0060x over previous
import jax
import jax.numpy as jnp
from jax.experimental import pallas as pl
from jax.experimental.pallas import tpu as pltpu

_CHUNK = 16


def _make_kernel(num_layers, hidden, in_size, seq_len, C):

    def fused_kernel(x_ref, w0_ref, wr_ref, b_ref, wd_ref, bd_ref, out_ref,
                     g0_scr, g1_scr, h0c_scr, h1c_scr, acc_scr, *state_scr):
        h_scrs = state_scr[0:num_layers]
        c_scrs = state_scr[num_layers:2 * num_layers]
        whb_scrs = state_scr[2 * num_layers:3 * num_layers]
        c = pl.program_id(0)
        nc = pl.num_programs(0) - 1

        @pl.when(c == 0)
        def _init():
            xs = x_ref[:, 0, :]
            g0_scr[0:seq_len, :] = (jnp.dot(xs, w0_ref[0:in_size, :],
                                            preferred_element_type=jnp.float32)
                                    + b_ref[0])
            g0_scr[seq_len:seq_len + C, :] = jnp.zeros((C, 4 * hidden),
                                                       jnp.float32)
            g1_scr[...] = jnp.zeros_like(g1_scr)
            whb_scrs[0][...] = w0_ref[in_size:in_size + hidden, :].astype(jnp.bfloat16)
            for l in range(1, num_layers):
                whb_scrs[l][...] = wr_ref[l - 1, hidden:2 * hidden, :].astype(jnp.bfloat16)
            for l in range(num_layers):
                h_scrs[l][...] = jnp.zeros_like(h_scrs[l])
                c_scrs[l][...] = jnp.zeros_like(c_scrs[l])
            acc_scr[...] = jnp.zeros_like(acc_scr)

        def apply_gates(gates, c_prev):
            i_g = jax.nn.sigmoid(gates[:, 0 * hidden:1 * hidden])
            f_g = jax.nn.sigmoid(gates[:, 1 * hidden:2 * hidden])
            g_g = jnp.tanh(gates[:, 2 * hidden:3 * hidden])
            o_g = jax.nn.sigmoid(gates[:, 3 * hidden:4 * hidden])
            c_new = f_g * c_prev + i_g * g_g
            h_new = o_g * jnp.tanh(c_new)
            return h_new, c_new

        base = c * C
        acc = acc_scr[...]
        g_chunk = g1_scr[...]
        g0_chunk = g0_scr[pl.ds(base, C), :]
        hs = [h_scrs[l][...] for l in range(num_layers)]
        cs = [c_scrs[l][...] for l in range(num_layers)]
        for j in range(C):
            gates0 = (g0_chunk[j:j + 1, :]
                      + jnp.dot(hs[0].astype(jnp.bfloat16), whb_scrs[0][...],
                                preferred_element_type=jnp.float32))
            gates1 = (g_chunk[j:j + 1, :]
                      + jnp.dot(hs[1].astype(jnp.bfloat16), whb_scrs[1][...],
                                preferred_element_type=jnp.float32))
            hs[0], cs[0] = apply_gates(gates0, cs[0])
            h0c_scr[j:j + 1, :] = hs[0]
            hs[1], cs[1] = apply_gates(gates1, cs[1])
            layer_in = hs[1]
            for l in range(2, num_layers):
                gl = (jnp.dot(layer_in, wr_ref[l - 1, 0:hidden, :],
                              preferred_element_type=jnp.float32)
                      + b_ref[l]
                      + jnp.dot(hs[l].astype(jnp.bfloat16), whb_scrs[l][...],
                                preferred_element_type=jnp.float32))
                hs[l], cs[l] = apply_gates(gl, cs[l])
                layer_in = hs[l]
            h1c_scr[0:1, pl.ds(j * hidden, hidden)] = layer_in
        for l in range(num_layers):
            h_scrs[l][...] = hs[l]
            c_scrs[l][...] = cs[l]
        acc_scr[...] = acc + jnp.dot(
            h1c_scr[...], wd_ref[...].reshape(C * hidden, -1),
            preferred_element_type=jnp.float32)

        g1_scr[...] = (jnp.dot(h0c_scr[...], wr_ref[0, 0:hidden, :],
                               preferred_element_type=jnp.float32)
                       + b_ref[1])

        @pl.when(c == nc)
        def _finalize():
            out_ref[...] = jax.nn.sigmoid(acc_scr[...] + bd_ref[...]).astype(out_ref.dtype)

    return fused_kernel


@jax.jit
def kernel(x, w0, wr, b_all, wd, bd):
    seq_len, _, in_size = x.shape
    num_layers = b_all.shape[0]
    hidden = b_all.shape[2] // 4
    out_size = wd.shape[2]
    lr = wr.shape[0]
    chunk = _CHUNK if seq_len % _CHUNK == 0 else 1
    nc = seq_len // chunk

    state_scratch = ([pltpu.VMEM((1, hidden), jnp.float32)
                      for _ in range(2 * num_layers)]
                     + [pltpu.VMEM((hidden, 4 * hidden), jnp.bfloat16)
                        for _ in range(num_layers)])

    return pl.pallas_call(
        _make_kernel(num_layers, hidden, in_size, seq_len, chunk),
        out_shape=jax.ShapeDtypeStruct((1, out_size), jnp.float32),
        grid_spec=pltpu.PrefetchScalarGridSpec(
            num_scalar_prefetch=0,
            grid=(nc + 1,),
            in_specs=[
                pl.BlockSpec((seq_len, 1, in_size), lambda c: (0, 0, 0)),
                pl.BlockSpec((in_size + hidden, 4 * hidden), lambda c: (0, 0)),
                pl.BlockSpec((lr, 2 * hidden, 4 * hidden), lambda c: (0, 0, 0)),
                pl.BlockSpec((num_layers, 1, 4 * hidden), lambda c: (0, 0, 0)),
                pl.BlockSpec((chunk, hidden, out_size),
                             lambda c: (jnp.maximum(c - 1, 0), 0, 0)),
                pl.BlockSpec((1, out_size), lambda c: (0, 0)),
            ],
            out_specs=pl.BlockSpec((1, out_size), lambda c: (0, 0)),
            scratch_shapes=[
                pltpu.VMEM((seq_len + chunk, 4 * hidden), jnp.float32),
                pltpu.VMEM((chunk, 4 * hidden), jnp.float32),
                pltpu.VMEM((chunk, hidden), jnp.float32),
                pltpu.VMEM((1, chunk * hidden), jnp.float32),
                pltpu.VMEM((1, out_size), jnp.float32),
            ] + state_scratch,
        ),
        compiler_params=pltpu.CompilerParams(
            dimension_semantics=("arbitrary",)),
    )(x, w0, wr, b_all, wd, bd)
